# baseline (speedup 1.0000x reference)
import functools

import jax
import jax.numpy as jnp
from jax import lax
from jax.experimental import pallas as pl
from jax.experimental.pallas import tpu as pltpu
from jax.experimental.pallas import tpu_sc as plsc

_TM = 128
_RT = 256


def _router_body(x_ref, w_ref, b_ref, logits_ref, eid_ref):
    logits = jnp.dot(x_ref[...], w_ref[...],
                     preferred_element_type=jnp.float32) + b_ref[...]
    logits_ref[...] = logits
    m = jnp.max(logits, axis=1, keepdims=True)
    col = lax.broadcasted_iota(jnp.int32, logits.shape, 1)
    eid_ref[...] = jnp.min(jnp.where(logits == m, col, logits.shape[1]),
                           axis=1, keepdims=True)


def _shared_body(x_ref, sw1_ref, sb1_ref, sw2_ref, sb2_ref, out_ref):
    x = x_ref[...]
    sb1 = sb1_ref[...]
    sb2 = sb2_ref[...]
    acc = jnp.zeros_like(out_ref)
    for e in range(sb1.shape[0]):
        h = jnp.maximum(
            jnp.dot(x, sw1_ref[e], preferred_element_type=jnp.float32)
            + sb1[e:e + 1, :], 0.0)
        acc = acc + jnp.dot(h, sw2_ref[e], preferred_element_type=jnp.float32)
        acc = acc + sb2[e:e + 1, :]
    out_ref[...] = acc


def _grouped_body(eids_ref, xs_ref, w1_ref, b1_ref, w2_ref, b2_ref, out_ref):
    del eids_ref
    h = jnp.maximum(
        jnp.dot(xs_ref[...], w1_ref[0],
                preferred_element_type=jnp.float32) + b1_ref[0], 0.0)
    out_ref[...] = (jnp.dot(h, w2_ref[0], preferred_element_type=jnp.float32)
                    + b2_ref[0])


def _sc_gather(table, idx):
    n_rows = idx.shape[0]
    d = table.shape[1]
    nw = 32
    b_per_w = n_rows // nw
    mesh = plsc.VectorSubcoreMesh(core_axis_name="c", subcore_axis_name="s")

    @functools.partial(
        pl.kernel, mesh=mesh,
        out_type=jax.ShapeDtypeStruct((n_rows, d), table.dtype),
        scratch_types=[pltpu.VMEM((b_per_w,), jnp.int32),
                       pltpu.VMEM((b_per_w, d), table.dtype),
                       pltpu.SemaphoreType.DMA])
    def k(table_hbm, idx_hbm, out_hbm, idx_v, rows_v, sem):
        wid = lax.axis_index("s") * 2 + lax.axis_index("c")
        base = wid * b_per_w
        pltpu.sync_copy(idx_hbm.at[pl.ds(base, b_per_w)], idx_v)
        pltpu.async_copy(table_hbm.at[idx_v], rows_v, sem).wait()
        pltpu.sync_copy(rows_v, out_hbm.at[pl.ds(base, b_per_w)])

    return k(table, idx)


def kernel(x, router_w, router_b, w1, b1, w2, b2, sw1, sb1, sw2, sb2):
    bsz, seq, hdim = x.shape
    flat = x.reshape(-1, hdim)
    t = flat.shape[0]
    n_exp = w1.shape[0]
    n_units = (t // _TM) + n_exp
    p = n_units * _TM

    logits, eid2 = pl.pallas_call(
        _router_body,
        grid=(t // _RT,),
        in_specs=[pl.BlockSpec((_RT, hdim), lambda i: (i, 0)),
                  pl.BlockSpec((hdim, n_exp), lambda i: (0, 0)),
                  pl.BlockSpec((1, n_exp), lambda i: (0, 0))],
        out_specs=[pl.BlockSpec((_RT, n_exp), lambda i: (i, 0)),
                   pl.BlockSpec((_RT, 1), lambda i: (i, 0))],
        out_shape=[jax.ShapeDtypeStruct((t, n_exp), jnp.float32),
                   jax.ShapeDtypeStruct((t, 1), jnp.int32)],
    )(flat, router_w, router_b.reshape(1, n_exp))
    eid = eid2[:, 0]

    onehot = (eid[:, None] == jnp.arange(n_exp, dtype=jnp.int32)[None, :]
              ).astype(jnp.int32)
    counts = onehot.sum(axis=0)
    rank = ((jnp.cumsum(onehot, axis=0) - onehot) * onehot).sum(axis=1)
    units_per = (counts + _TM - 1) // _TM
    cum_units = jnp.cumsum(units_per)
    unit_start = cum_units - units_per
    pos = unit_start[eid] * _TM + rank
    sort_idx = jnp.zeros((p,), jnp.int32).at[pos].set(
        jnp.arange(t, dtype=jnp.int32))
    uids = jnp.arange(n_units, dtype=jnp.int32)
    unit_eid_raw = (uids[:, None] >= cum_units[None, :]).astype(jnp.int32).sum(1)
    last_eid = unit_eid_raw[jnp.sum(units_per) - 1]
    unit_eid = jnp.minimum(unit_eid_raw, last_eid)

    xs = _sc_gather(flat, sort_idx)

    shared = pl.pallas_call(
        _shared_body,
        grid=(t // _RT,),
        in_specs=[pl.BlockSpec((_RT, hdim), lambda i: (i, 0)),
                  pl.BlockSpec(sw1.shape, lambda i: (0, 0, 0)),
                  pl.BlockSpec(sb1.shape, lambda i: (0, 0)),
                  pl.BlockSpec(sw2.shape, lambda i: (0, 0, 0)),
                  pl.BlockSpec(sb2.shape, lambda i: (0, 0))],
        out_specs=pl.BlockSpec((_RT, hdim), lambda i: (i, 0)),
        out_shape=jax.ShapeDtypeStruct((t, hdim), jnp.float32),
    )(flat, sw1, sb1, sw2, sb2)

    grid_spec = pltpu.PrefetchScalarGridSpec(
        num_scalar_prefetch=1,
        grid=(n_units,),
        in_specs=[pl.BlockSpec((_TM, hdim), lambda u, eids: (u, 0)),
                  pl.BlockSpec((1, hdim, hdim), lambda u, eids: (eids[u], 0, 0)),
                  pl.BlockSpec((1, 1, hdim), lambda u, eids: (eids[u], 0, 0)),
                  pl.BlockSpec((1, hdim, hdim), lambda u, eids: (eids[u], 0, 0)),
                  pl.BlockSpec((1, 1, hdim), lambda u, eids: (eids[u], 0, 0))],
        out_specs=pl.BlockSpec((_TM, hdim), lambda u, eids: (u, 0)),
    )
    ys = pl.pallas_call(
        _grouped_body, grid_spec=grid_spec,
        out_shape=jax.ShapeDtypeStruct((p, hdim), jnp.float32),
    )(unit_eid, xs, w1, b1.reshape(n_exp, 1, hdim), w2,
      b2.reshape(n_exp, 1, hdim))

    routed = _sc_gather(ys, pos)
    out = (shared + routed).reshape(bsz, seq, hdim)
    return (out, logits)


# speedup vs baseline: 1.5734x; 1.5734x over previous
import functools

import jax
import jax.numpy as jnp
from jax import lax
from jax.experimental import pallas as pl
from jax.experimental.pallas import tpu as pltpu
from jax.experimental.pallas import tpu_sc as plsc

_TM = 128
_RT = 256


def _router_body(x_ref, w_ref, b_ref, logits_ref, eid_ref):
    logits = jnp.dot(x_ref[...], w_ref[...],
                     preferred_element_type=jnp.float32) + b_ref[...]
    logits_ref[...] = logits
    m = jnp.max(logits, axis=1, keepdims=True)
    col = lax.broadcasted_iota(jnp.int32, logits.shape, 1)
    eid_ref[...] = jnp.min(jnp.where(logits == m, col, logits.shape[1]),
                           axis=1, keepdims=True)


def _shared_body(x_ref, sw1_ref, sb1_ref, sw2_ref, sb2_ref, out_ref):
    x = x_ref[...]
    sb1 = sb1_ref[...]
    sb2 = sb2_ref[...]
    acc = jnp.zeros_like(out_ref)
    for e in range(sb1.shape[0]):
        h = jnp.maximum(
            jnp.dot(x, sw1_ref[e], preferred_element_type=jnp.float32)
            + sb1[e:e + 1, :], 0.0)
        acc = acc + jnp.dot(h, sw2_ref[e], preferred_element_type=jnp.float32)
        acc = acc + sb2[e:e + 1, :]
    out_ref[...] = acc


def _grouped_body(eids_ref, xs_ref, w1_ref, b1_ref, w2_ref, b2_ref, out_ref):
    del eids_ref
    h = jnp.maximum(
        jnp.dot(xs_ref[...], w1_ref[0],
                preferred_element_type=jnp.float32) + b1_ref[0], 0.0)
    out_ref[...] = (jnp.dot(h, w2_ref[0], preferred_element_type=jnp.float32)
                    + b2_ref[0])


def _sc_gather(table, idx):
    n_rows = idx.shape[0]
    d = table.shape[1]
    nw = 32
    b_per_w = n_rows // nw
    mesh = plsc.VectorSubcoreMesh(core_axis_name="c", subcore_axis_name="s")

    @functools.partial(
        pl.kernel, mesh=mesh,
        out_type=jax.ShapeDtypeStruct((n_rows, d), table.dtype),
        scratch_types=[pltpu.VMEM((b_per_w,), jnp.int32),
                       pltpu.VMEM((b_per_w, d), table.dtype),
                       pltpu.SemaphoreType.DMA])
    def k(table_hbm, idx_hbm, out_hbm, idx_v, rows_v, sem):
        wid = lax.axis_index("s") * 2 + lax.axis_index("c")
        base = wid * b_per_w
        pltpu.sync_copy(idx_hbm.at[pl.ds(base, b_per_w)], idx_v)
        pltpu.async_copy(table_hbm.at[idx_v], rows_v, sem).wait()
        pltpu.sync_copy(rows_v, out_hbm.at[pl.ds(base, b_per_w)])

    return k(table, idx)


def kernel(x, router_w, router_b, w1, b1, w2, b2, sw1, sb1, sw2, sb2):
    bsz, seq, hdim = x.shape
    flat = x.reshape(-1, hdim)
    t = flat.shape[0]
    n_exp = w1.shape[0]
    n_units = (t // _TM) + n_exp
    p = n_units * _TM

    logits, eid2 = pl.pallas_call(
        _router_body,
        grid=(t // _RT,),
        in_specs=[pl.BlockSpec((_RT, hdim), lambda i: (i, 0)),
                  pl.BlockSpec((hdim, n_exp), lambda i: (0, 0)),
                  pl.BlockSpec((1, n_exp), lambda i: (0, 0))],
        out_specs=[pl.BlockSpec((_RT, n_exp), lambda i: (i, 0)),
                   pl.BlockSpec((_RT, 1), lambda i: (i, 0))],
        out_shape=[jax.ShapeDtypeStruct((t, n_exp), jnp.float32),
                   jax.ShapeDtypeStruct((t, 1), jnp.int32)],
    )(flat, router_w, router_b.reshape(1, n_exp))
    eid = eid2[:, 0]

    onehot = (eid[:, None] == jnp.arange(n_exp, dtype=jnp.int32)[None, :]
              ).astype(jnp.int32)
    counts = onehot.sum(axis=0)
    rank = ((jnp.cumsum(onehot, axis=0) - onehot) * onehot).sum(axis=1)
    units_per = (counts + _TM - 1) // _TM
    cum_units = jnp.cumsum(units_per)
    unit_start = cum_units - units_per
    pos = unit_start[eid] * _TM + rank
    sort_idx = (jnp.arange(p, dtype=jnp.int32) % t).at[pos].set(
        jnp.arange(t, dtype=jnp.int32))
    uids = jnp.arange(n_units, dtype=jnp.int32)
    unit_eid_raw = (uids[:, None] >= cum_units[None, :]).astype(jnp.int32).sum(1)
    last_eid = unit_eid_raw[jnp.sum(units_per) - 1]
    unit_eid = jnp.minimum(unit_eid_raw, last_eid)

    xs = _sc_gather(flat, sort_idx)

    shared = pl.pallas_call(
        _shared_body,
        grid=(t // _RT,),
        in_specs=[pl.BlockSpec((_RT, hdim), lambda i: (i, 0)),
                  pl.BlockSpec(sw1.shape, lambda i: (0, 0, 0)),
                  pl.BlockSpec(sb1.shape, lambda i: (0, 0)),
                  pl.BlockSpec(sw2.shape, lambda i: (0, 0, 0)),
                  pl.BlockSpec(sb2.shape, lambda i: (0, 0))],
        out_specs=pl.BlockSpec((_RT, hdim), lambda i: (i, 0)),
        out_shape=jax.ShapeDtypeStruct((t, hdim), jnp.float32),
    )(flat, sw1, sb1, sw2, sb2)

    grid_spec = pltpu.PrefetchScalarGridSpec(
        num_scalar_prefetch=1,
        grid=(n_units,),
        in_specs=[pl.BlockSpec((_TM, hdim), lambda u, eids: (u, 0)),
                  pl.BlockSpec((1, hdim, hdim), lambda u, eids: (eids[u], 0, 0)),
                  pl.BlockSpec((1, 1, hdim), lambda u, eids: (eids[u], 0, 0)),
                  pl.BlockSpec((1, hdim, hdim), lambda u, eids: (eids[u], 0, 0)),
                  pl.BlockSpec((1, 1, hdim), lambda u, eids: (eids[u], 0, 0))],
        out_specs=pl.BlockSpec((_TM, hdim), lambda u, eids: (u, 0)),
    )
    ys = pl.pallas_call(
        _grouped_body, grid_spec=grid_spec,
        out_shape=jax.ShapeDtypeStruct((p, hdim), jnp.float32),
    )(unit_eid, xs, w1, b1.reshape(n_exp, 1, hdim), w2,
      b2.reshape(n_exp, 1, hdim))

    routed = _sc_gather(ys, pos)
    out = (shared + routed).reshape(bsz, seq, hdim)
    return (out, logits)


# speedup vs baseline: 1.7195x; 1.0929x over previous
import functools

import jax
import jax.numpy as jnp
from jax import lax
from jax.experimental import pallas as pl
from jax.experimental.pallas import tpu as pltpu
from jax.experimental.pallas import tpu_sc as plsc

_TM = 128
_RT = 256


def _router_body(x_ref, w_ref, b_ref, logits_ref, eid_ref):
    logits = jnp.dot(x_ref[...], w_ref[...],
                     preferred_element_type=jnp.float32) + b_ref[...]
    logits_ref[...] = logits
    m = jnp.max(logits, axis=1, keepdims=True)
    col = lax.broadcasted_iota(jnp.int32, logits.shape, 1)
    eid_ref[...] = jnp.min(jnp.where(logits == m, col, logits.shape[1]),
                           axis=1, keepdims=True)


def _shared_body(x_ref, sw1_ref, sb1_ref, sw2_ref, sb2_ref, out_ref):
    x = x_ref[...]
    sb1 = sb1_ref[...]
    sb2 = sb2_ref[...]
    acc = jnp.zeros_like(out_ref)
    for e in range(sb1.shape[0]):
        h = jnp.maximum(
            jnp.dot(x, sw1_ref[e], preferred_element_type=jnp.float32)
            + sb1[e:e + 1, :], 0.0)
        acc = acc + jnp.dot(h, sw2_ref[e], preferred_element_type=jnp.float32)
        acc = acc + sb2[e:e + 1, :]
    out_ref[...] = acc


def _grouped_body(eids_ref, xs_ref, w1_ref, b1_ref, w2_ref, b2_ref, out_ref):
    del eids_ref
    h = jnp.maximum(
        jnp.dot(xs_ref[...], w1_ref[0],
                preferred_element_type=jnp.float32) + b1_ref[0], 0.0)
    out_ref[...] = (jnp.dot(h, w2_ref[0], preferred_element_type=jnp.float32)
                    + b2_ref[0])


def _sc_scatter_rows(rows, idx2d, out_rows):
    t, d = rows.shape
    nw, b_per_w = idx2d.shape
    mesh = plsc.VectorSubcoreMesh(core_axis_name="c", subcore_axis_name="s")

    @functools.partial(
        pl.kernel, mesh=mesh,
        out_type=jax.ShapeDtypeStruct((out_rows, d), rows.dtype),
        scratch_types=[pltpu.VMEM((1, b_per_w), jnp.int32),
                       pltpu.VMEM((b_per_w, d), rows.dtype),
                       pltpu.SemaphoreType.DMA])
    def k(rows_hbm, idx_hbm, out_hbm, idx_v, rows_v, sem):
        wid = lax.axis_index("s") * 2 + lax.axis_index("c")
        pltpu.sync_copy(idx_hbm.at[pl.ds(wid, 1)], idx_v)
        pltpu.sync_copy(rows_hbm.at[pl.ds(wid * b_per_w, b_per_w)], rows_v)
        pltpu.async_copy(rows_v, out_hbm.at[idx_v.at[0]], sem).wait()

    return k(rows, idx2d)


def _sc_gather(table, idx):
    n_rows = idx.shape[0]
    d = table.shape[1]
    nw = 32
    b_per_w = n_rows // nw
    mesh = plsc.VectorSubcoreMesh(core_axis_name="c", subcore_axis_name="s")

    @functools.partial(
        pl.kernel, mesh=mesh,
        out_type=jax.ShapeDtypeStruct((n_rows, d), table.dtype),
        scratch_types=[pltpu.VMEM((b_per_w,), jnp.int32),
                       pltpu.VMEM((b_per_w, d), table.dtype),
                       pltpu.SemaphoreType.DMA])
    def k(table_hbm, idx_hbm, out_hbm, idx_v, rows_v, sem):
        wid = lax.axis_index("s") * 2 + lax.axis_index("c")
        base = wid * b_per_w
        pltpu.sync_copy(idx_hbm.at[pl.ds(base, b_per_w)], idx_v)
        pltpu.async_copy(table_hbm.at[idx_v], rows_v, sem).wait()
        pltpu.sync_copy(rows_v, out_hbm.at[pl.ds(base, b_per_w)])

    return k(table, idx)


def kernel(x, router_w, router_b, w1, b1, w2, b2, sw1, sb1, sw2, sb2):
    bsz, seq, hdim = x.shape
    flat = x.reshape(-1, hdim)
    t = flat.shape[0]
    n_exp = w1.shape[0]
    n_units = (t // _TM) + n_exp
    p = n_units * _TM

    logits, eid2 = pl.pallas_call(
        _router_body,
        grid=(t // _RT,),
        in_specs=[pl.BlockSpec((_RT, hdim), lambda i: (i, 0)),
                  pl.BlockSpec((hdim, n_exp), lambda i: (0, 0)),
                  pl.BlockSpec((1, n_exp), lambda i: (0, 0))],
        out_specs=[pl.BlockSpec((_RT, n_exp), lambda i: (i, 0)),
                   pl.BlockSpec((_RT, 1), lambda i: (i, 0))],
        out_shape=[jax.ShapeDtypeStruct((t, n_exp), jnp.float32),
                   jax.ShapeDtypeStruct((t, 1), jnp.int32)],
    )(flat, router_w, router_b.reshape(1, n_exp))
    eid = eid2[:, 0]

    onehot = (eid[:, None] == jnp.arange(n_exp, dtype=jnp.int32)[None, :]
              ).astype(jnp.int32)
    counts = onehot.sum(axis=0)
    rank = ((jnp.cumsum(onehot, axis=0) - onehot) * onehot).sum(axis=1)
    units_per = (counts + _TM - 1) // _TM
    cum_units = jnp.cumsum(units_per)
    unit_start = cum_units - units_per
    pos = unit_start[eid] * _TM + rank
    uids = jnp.arange(n_units, dtype=jnp.int32)
    unit_eid_raw = (uids[:, None] >= cum_units[None, :]).astype(jnp.int32).sum(1)
    last_eid = unit_eid_raw[jnp.sum(units_per) - 1]
    unit_eid = jnp.minimum(unit_eid_raw, last_eid)

    xs = _sc_scatter_rows(flat, pos.reshape(32, t // 32), p)

    shared = pl.pallas_call(
        _shared_body,
        grid=(t // _RT,),
        in_specs=[pl.BlockSpec((_RT, hdim), lambda i: (i, 0)),
                  pl.BlockSpec(sw1.shape, lambda i: (0, 0, 0)),
                  pl.BlockSpec(sb1.shape, lambda i: (0, 0)),
                  pl.BlockSpec(sw2.shape, lambda i: (0, 0, 0)),
                  pl.BlockSpec(sb2.shape, lambda i: (0, 0))],
        out_specs=pl.BlockSpec((_RT, hdim), lambda i: (i, 0)),
        out_shape=jax.ShapeDtypeStruct((t, hdim), jnp.float32),
    )(flat, sw1, sb1, sw2, sb2)

    grid_spec = pltpu.PrefetchScalarGridSpec(
        num_scalar_prefetch=1,
        grid=(n_units,),
        in_specs=[pl.BlockSpec((_TM, hdim), lambda u, eids: (u, 0)),
                  pl.BlockSpec((1, hdim, hdim), lambda u, eids: (eids[u], 0, 0)),
                  pl.BlockSpec((1, 1, hdim), lambda u, eids: (eids[u], 0, 0)),
                  pl.BlockSpec((1, hdim, hdim), lambda u, eids: (eids[u], 0, 0)),
                  pl.BlockSpec((1, 1, hdim), lambda u, eids: (eids[u], 0, 0))],
        out_specs=pl.BlockSpec((_TM, hdim), lambda u, eids: (u, 0)),
    )
    ys = pl.pallas_call(
        _grouped_body, grid_spec=grid_spec,
        out_shape=jax.ShapeDtypeStruct((p, hdim), jnp.float32),
    )(unit_eid, xs, w1, b1.reshape(n_exp, 1, hdim), w2,
      b2.reshape(n_exp, 1, hdim))

    routed = _sc_gather(ys, pos)
    out = (shared + routed).reshape(bsz, seq, hdim)
    return (out, logits)


# speedup vs baseline: 2.0824x; 1.2110x over previous
import functools

import jax
import jax.numpy as jnp
from jax import lax
from jax.experimental import pallas as pl
from jax.experimental.pallas import tpu as pltpu
from jax.experimental.pallas import tpu_sc as plsc

_TM = 128
_RT = 256


def _router_body(x_ref, w_ref, b_ref, logits_ref, pos_ref, unit_eid_ref):
    t, e = logits_ref.shape
    nu = unit_eid_ref.shape[0]
    ck = 256
    logits = jnp.dot(x_ref[...], w_ref[...],
                     preferred_element_type=jnp.float32) + b_ref[...]
    logits_ref[...] = logits
    m = jnp.max(logits, axis=1, keepdims=True)
    col = lax.broadcasted_iota(jnp.int32, (t, e), 1)
    eid = jnp.min(jnp.where(logits == m, col, e), axis=1, keepdims=True)
    oh = (col == eid).astype(jnp.float32)
    tri = (lax.broadcasted_iota(jnp.int32, (ck, ck), 1) <=
           lax.broadcasted_iota(jnp.int32, (ck, ck), 0)).astype(jnp.float32)
    carry = jnp.zeros((1, e), jnp.float32)
    chunks = []
    for i in range(t // ck):
        c = jnp.dot(tri, oh[i * ck:(i + 1) * ck],
                    preferred_element_type=jnp.float32) + carry
        chunks.append(c)
        carry = c[ck - 1:ck, :]
    cum = jnp.concatenate(chunks, axis=0)
    counts = carry
    rank = jnp.sum((cum - oh) * oh, axis=1, keepdims=True)
    units_per = jnp.floor((counts + (_TM - 1)) * (1.0 / _TM))
    triu = (lax.broadcasted_iota(jnp.int32, (e, e), 0) <=
            lax.broadcasted_iota(jnp.int32, (e, e), 1)).astype(jnp.float32)
    cum_units = jnp.dot(units_per, triu, preferred_element_type=jnp.float32)
    unit_start = cum_units - units_per
    pos = (jnp.sum(oh * unit_start, axis=1, keepdims=True) * _TM + rank)
    pos_ref[...] = pos.astype(jnp.int32)
    u_iota = lax.broadcasted_iota(jnp.int32, (nu, 1), 0).astype(jnp.float32)
    raw = jnp.sum((u_iota >= cum_units).astype(jnp.float32), axis=1,
                  keepdims=True)
    total_units = jnp.sum(units_per, axis=1, keepdims=True)
    last_eid = jnp.sum(jnp.where(u_iota == total_units - 1.0, raw, 0.0),
                       axis=0, keepdims=True)
    unit_eid_ref[...] = jnp.minimum(raw, last_eid).astype(jnp.int32)


def _shared_body(x_ref, sw1_ref, sb1_ref, sw2_ref, sb2_ref, out_ref):
    x = x_ref[...]
    sb1 = sb1_ref[...]
    sb2 = sb2_ref[...]
    acc = jnp.zeros_like(out_ref)
    for e in range(sb1.shape[0]):
        h = jnp.maximum(
            jnp.dot(x, sw1_ref[e], preferred_element_type=jnp.float32)
            + sb1[e:e + 1, :], 0.0)
        acc = acc + jnp.dot(h, sw2_ref[e], preferred_element_type=jnp.float32)
        acc = acc + sb2[e:e + 1, :]
    out_ref[...] = acc


def _grouped_body(eids_ref, xs_ref, w1_ref, b1_ref, w2_ref, b2_ref, out_ref):
    del eids_ref
    h = jnp.maximum(
        jnp.dot(xs_ref[...], w1_ref[0],
                preferred_element_type=jnp.float32) + b1_ref[0], 0.0)
    out_ref[...] = (jnp.dot(h, w2_ref[0], preferred_element_type=jnp.float32)
                    + b2_ref[0])


def _sc_scatter_rows(rows, idx2d, out_rows):
    t, d = rows.shape
    nw, b_per_w = idx2d.shape
    mesh = plsc.VectorSubcoreMesh(core_axis_name="c", subcore_axis_name="s")

    @functools.partial(
        pl.kernel, mesh=mesh,
        out_type=jax.ShapeDtypeStruct((out_rows, d), rows.dtype),
        scratch_types=[pltpu.VMEM((1, b_per_w), jnp.int32),
                       pltpu.VMEM((b_per_w, d), rows.dtype),
                       pltpu.SemaphoreType.DMA])
    def k(rows_hbm, idx_hbm, out_hbm, idx_v, rows_v, sem):
        wid = lax.axis_index("s") * 2 + lax.axis_index("c")
        pltpu.sync_copy(idx_hbm.at[pl.ds(wid, 1)], idx_v)
        pltpu.sync_copy(rows_hbm.at[pl.ds(wid * b_per_w, b_per_w)], rows_v)
        pltpu.async_copy(rows_v, out_hbm.at[idx_v.at[0]], sem).wait()

    return k(rows, idx2d)


def _sc_gather(table, idx):
    n_rows = idx.shape[0]
    d = table.shape[1]
    nw = 32
    b_per_w = n_rows // nw
    mesh = plsc.VectorSubcoreMesh(core_axis_name="c", subcore_axis_name="s")

    @functools.partial(
        pl.kernel, mesh=mesh,
        out_type=jax.ShapeDtypeStruct((n_rows, d), table.dtype),
        scratch_types=[pltpu.VMEM((b_per_w,), jnp.int32),
                       pltpu.VMEM((b_per_w, d), table.dtype),
                       pltpu.SemaphoreType.DMA])
    def k(table_hbm, idx_hbm, out_hbm, idx_v, rows_v, sem):
        wid = lax.axis_index("s") * 2 + lax.axis_index("c")
        base = wid * b_per_w
        pltpu.sync_copy(idx_hbm.at[pl.ds(base, b_per_w)], idx_v)
        pltpu.async_copy(table_hbm.at[idx_v], rows_v, sem).wait()
        pltpu.sync_copy(rows_v, out_hbm.at[pl.ds(base, b_per_w)])

    return k(table, idx)


def kernel(x, router_w, router_b, w1, b1, w2, b2, sw1, sb1, sw2, sb2):
    bsz, seq, hdim = x.shape
    flat = x.reshape(-1, hdim)
    t = flat.shape[0]
    n_exp = w1.shape[0]
    n_units = (t // _TM) + n_exp
    p = n_units * _TM

    logits, pos2, unit_eid2 = pl.pallas_call(
        _router_body,
        grid=(1,),
        in_specs=[pl.BlockSpec((t, hdim), lambda i: (0, 0)),
                  pl.BlockSpec((hdim, n_exp), lambda i: (0, 0)),
                  pl.BlockSpec((1, n_exp), lambda i: (0, 0))],
        out_specs=[pl.BlockSpec((t, n_exp), lambda i: (0, 0)),
                   pl.BlockSpec((t, 1), lambda i: (0, 0)),
                   pl.BlockSpec((n_units, 1), lambda i: (0, 0))],
        out_shape=[jax.ShapeDtypeStruct((t, n_exp), jnp.float32),
                   jax.ShapeDtypeStruct((t, 1), jnp.int32),
                   jax.ShapeDtypeStruct((n_units, 1), jnp.int32)],
    )(flat, router_w, router_b.reshape(1, n_exp))
    pos = pos2.reshape(t)
    unit_eid = unit_eid2.reshape(n_units)

    xs = _sc_scatter_rows(flat, pos.reshape(32, t // 32), p)

    shared = pl.pallas_call(
        _shared_body,
        grid=(t // _RT,),
        in_specs=[pl.BlockSpec((_RT, hdim), lambda i: (i, 0)),
                  pl.BlockSpec(sw1.shape, lambda i: (0, 0, 0)),
                  pl.BlockSpec(sb1.shape, lambda i: (0, 0)),
                  pl.BlockSpec(sw2.shape, lambda i: (0, 0, 0)),
                  pl.BlockSpec(sb2.shape, lambda i: (0, 0))],
        out_specs=pl.BlockSpec((_RT, hdim), lambda i: (i, 0)),
        out_shape=jax.ShapeDtypeStruct((t, hdim), jnp.float32),
    )(flat, sw1, sb1, sw2, sb2)

    grid_spec = pltpu.PrefetchScalarGridSpec(
        num_scalar_prefetch=1,
        grid=(n_units,),
        in_specs=[pl.BlockSpec((_TM, hdim), lambda u, eids: (u, 0)),
                  pl.BlockSpec((1, hdim, hdim), lambda u, eids: (eids[u], 0, 0)),
                  pl.BlockSpec((1, 1, hdim), lambda u, eids: (eids[u], 0, 0)),
                  pl.BlockSpec((1, hdim, hdim), lambda u, eids: (eids[u], 0, 0)),
                  pl.BlockSpec((1, 1, hdim), lambda u, eids: (eids[u], 0, 0))],
        out_specs=pl.BlockSpec((_TM, hdim), lambda u, eids: (u, 0)),
    )
    ys = pl.pallas_call(
        _grouped_body, grid_spec=grid_spec,
        out_shape=jax.ShapeDtypeStruct((p, hdim), jnp.float32),
    )(unit_eid, xs, w1, b1.reshape(n_exp, 1, hdim), w2,
      b2.reshape(n_exp, 1, hdim))

    routed = _sc_gather(ys, pos)
    out = (shared + routed).reshape(bsz, seq, hdim)
    return (out, logits)


# speedup vs baseline: 2.5121x; 1.2064x over previous
import functools

import jax
import jax.numpy as jnp
from jax import lax
from jax.experimental import pallas as pl
from jax.experimental.pallas import tpu as pltpu
from jax.experimental.pallas import tpu_sc as plsc

_TM = 128


def _router_body(x_ref, w_ref, b_ref, logits_ref, pos_ref, unit_start_ref,
                 units_per_ref):
    t, e = logits_ref.shape
    nu = 0
    ck = 256
    logits = jnp.dot(x_ref[...], w_ref[...],
                     preferred_element_type=jnp.float32) + b_ref[...]
    logits_ref[...] = logits
    m = jnp.max(logits, axis=1, keepdims=True)
    col = lax.broadcasted_iota(jnp.int32, (t, e), 1)
    eid = jnp.min(jnp.where(logits == m, col, e), axis=1, keepdims=True)
    oh = (col == eid).astype(jnp.float32)
    tri = (lax.broadcasted_iota(jnp.int32, (ck, ck), 1) <=
           lax.broadcasted_iota(jnp.int32, (ck, ck), 0)).astype(jnp.float32)
    carry = jnp.zeros((1, e), jnp.float32)
    chunks = []
    for i in range(t // ck):
        c = jnp.dot(tri, oh[i * ck:(i + 1) * ck],
                    preferred_element_type=jnp.float32) + carry
        chunks.append(c)
        carry = c[ck - 1:ck, :]
    cum = jnp.concatenate(chunks, axis=0)
    counts = carry
    rank = jnp.sum((cum - oh) * oh, axis=1, keepdims=True)
    units_per = jnp.floor((counts + (_TM - 1)) * (1.0 / _TM))
    triu = (lax.broadcasted_iota(jnp.int32, (e, e), 0) <=
            lax.broadcasted_iota(jnp.int32, (e, e), 1)).astype(jnp.float32)
    cum_units = jnp.dot(units_per, triu, preferred_element_type=jnp.float32)
    unit_start = cum_units - units_per
    pos = (jnp.sum(oh * unit_start, axis=1, keepdims=True) * _TM + rank)
    pos_ref[...] = pos.astype(jnp.int32)
    del nu
    unit_start_ref[...] = unit_start.astype(jnp.int32)
    units_per_ref[...] = units_per.astype(jnp.int32)


def _moe_body(start_ref, nunit_ref, xs_ref, w1_ref, b1_ref, w2_ref, b2_ref,
              sw1_ref, sb1_ref, sw2_ref, sb2_ref, ys_ref):
    e = pl.program_id(0)
    s0 = start_ref[e]

    def unit(j, carry):
        blk = xs_ref[pl.ds(j * _TM, _TM), :]
        h = jnp.maximum(
            jnp.dot(blk, w1_ref[0], preferred_element_type=jnp.float32)
            + b1_ref[0], 0.0)
        y = jnp.dot(h, w2_ref[0], preferred_element_type=jnp.float32) + b2_ref[0]
        for s in range(sb1_ref.shape[0]):
            hs = jnp.maximum(
                jnp.dot(blk, sw1_ref[s], preferred_element_type=jnp.float32)
                + sb1_ref[s], 0.0)
            y = y + jnp.dot(hs, sw2_ref[s],
                            preferred_element_type=jnp.float32) + sb2_ref[s]
        ys_ref[pl.ds(j * _TM, _TM), :] = y
        return carry

    lax.fori_loop(s0, s0 + nunit_ref[e], unit, 0)


def _sc_scatter_rows(rows, idx2d, out_rows):
    t, d = rows.shape
    nw, b_per_w = idx2d.shape
    mesh = plsc.VectorSubcoreMesh(core_axis_name="c", subcore_axis_name="s")

    @functools.partial(
        pl.kernel, mesh=mesh,
        out_type=jax.ShapeDtypeStruct((out_rows, d), rows.dtype),
        scratch_types=[pltpu.VMEM((1, b_per_w), jnp.int32),
                       pltpu.VMEM((b_per_w, d), rows.dtype),
                       pltpu.SemaphoreType.DMA])
    def k(rows_hbm, idx_hbm, out_hbm, idx_v, rows_v, sem):
        wid = lax.axis_index("s") * 2 + lax.axis_index("c")
        pltpu.sync_copy(idx_hbm.at[pl.ds(wid, 1)], idx_v)
        pltpu.sync_copy(rows_hbm.at[pl.ds(wid * b_per_w, b_per_w)], rows_v)
        pltpu.async_copy(rows_v, out_hbm.at[idx_v.at[0]], sem).wait()

    return k(rows, idx2d)


def _sc_gather(table, idx):
    n_rows = idx.shape[0]
    d = table.shape[1]
    nw = 32
    b_per_w = n_rows // nw
    mesh = plsc.VectorSubcoreMesh(core_axis_name="c", subcore_axis_name="s")

    @functools.partial(
        pl.kernel, mesh=mesh,
        out_type=jax.ShapeDtypeStruct((n_rows, d), table.dtype),
        scratch_types=[pltpu.VMEM((b_per_w,), jnp.int32),
                       pltpu.VMEM((b_per_w, d), table.dtype),
                       pltpu.SemaphoreType.DMA])
    def k(table_hbm, idx_hbm, out_hbm, idx_v, rows_v, sem):
        wid = lax.axis_index("s") * 2 + lax.axis_index("c")
        base = wid * b_per_w
        pltpu.sync_copy(idx_hbm.at[pl.ds(base, b_per_w)], idx_v)
        pltpu.async_copy(table_hbm.at[idx_v], rows_v, sem).wait()
        pltpu.sync_copy(rows_v, out_hbm.at[pl.ds(base, b_per_w)])

    return k(table, idx)


def kernel(x, router_w, router_b, w1, b1, w2, b2, sw1, sb1, sw2, sb2):
    bsz, seq, hdim = x.shape
    flat = x.reshape(-1, hdim)
    t = flat.shape[0]
    n_exp = w1.shape[0]
    n_units = (t // _TM) + n_exp
    p = n_units * _TM

    logits, pos2, unit_start, units_per = pl.pallas_call(
        _router_body,
        grid=(1,),
        in_specs=[pl.BlockSpec((t, hdim), lambda i: (0, 0)),
                  pl.BlockSpec((hdim, n_exp), lambda i: (0, 0)),
                  pl.BlockSpec((1, n_exp), lambda i: (0, 0))],
        out_specs=[pl.BlockSpec((t, n_exp), lambda i: (0, 0)),
                   pl.BlockSpec((t, 1), lambda i: (0, 0)),
                   pl.BlockSpec((1, n_exp), lambda i: (0, 0)),
                   pl.BlockSpec((1, n_exp), lambda i: (0, 0))],
        out_shape=[jax.ShapeDtypeStruct((t, n_exp), jnp.float32),
                   jax.ShapeDtypeStruct((t, 1), jnp.int32),
                   jax.ShapeDtypeStruct((1, n_exp), jnp.int32),
                   jax.ShapeDtypeStruct((1, n_exp), jnp.int32)],
    )(flat, router_w, router_b.reshape(1, n_exp))
    pos = pos2.reshape(t)

    xs = _sc_scatter_rows(flat, pos.reshape(32, t // 32), p)

    grid_spec = pltpu.PrefetchScalarGridSpec(
        num_scalar_prefetch=2,
        grid=(n_exp,),
        in_specs=[pl.BlockSpec((p, hdim), lambda e, s, n: (0, 0)),
                  pl.BlockSpec((1, hdim, hdim), lambda e, s, n: (e, 0, 0)),
                  pl.BlockSpec((1, 1, hdim), lambda e, s, n: (e, 0, 0)),
                  pl.BlockSpec((1, hdim, hdim), lambda e, s, n: (e, 0, 0)),
                  pl.BlockSpec((1, 1, hdim), lambda e, s, n: (e, 0, 0)),
                  pl.BlockSpec(sw1.shape, lambda e, s, n: (0, 0, 0)),
                  pl.BlockSpec((sb1.shape[0], 1, hdim), lambda e, s, n: (0, 0, 0)),
                  pl.BlockSpec(sw2.shape, lambda e, s, n: (0, 0, 0)),
                  pl.BlockSpec((sb2.shape[0], 1, hdim), lambda e, s, n: (0, 0, 0))],
        out_specs=pl.BlockSpec((p, hdim), lambda e, s, n: (0, 0)),
    )
    nsh = sw1.shape[0]
    ys = pl.pallas_call(
        _moe_body, grid_spec=grid_spec,
        out_shape=jax.ShapeDtypeStruct((p, hdim), jnp.float32),
    )(unit_start.reshape(n_exp), units_per.reshape(n_exp),
      xs, w1, b1.reshape(n_exp, 1, hdim), w2, b2.reshape(n_exp, 1, hdim),
      sw1, sb1.reshape(nsh, 1, hdim), sw2, sb2.reshape(nsh, 1, hdim))

    out = _sc_gather(ys, pos).reshape(bsz, seq, hdim)
    return (out, logits)


# speedup vs baseline: 2.6117x; 1.0396x over previous
import functools

import jax
import jax.numpy as jnp
from jax import lax
from jax.experimental import pallas as pl
from jax.experimental.pallas import tpu as pltpu
from jax.experimental.pallas import tpu_sc as plsc

_TM = 128


def _router_body(x_ref, w_ref, b_ref, logits_ref, pos_ref, unit_start_ref,
                 units_per_ref):
    t, e = logits_ref.shape
    nu = 0
    ck = 256
    logits = jnp.dot(x_ref[...], w_ref[...],
                     preferred_element_type=jnp.float32) + b_ref[...]
    logits_ref[...] = logits
    m = jnp.max(logits, axis=1, keepdims=True)
    col = lax.broadcasted_iota(jnp.int32, (t, e), 1)
    eid = jnp.min(jnp.where(logits == m, col, e), axis=1, keepdims=True)
    oh = (col == eid).astype(jnp.float32)
    tri = (lax.broadcasted_iota(jnp.int32, (ck, ck), 1) <=
           lax.broadcasted_iota(jnp.int32, (ck, ck), 0)).astype(jnp.float32)
    carry = jnp.zeros((1, e), jnp.float32)
    chunks = []
    for i in range(t // ck):
        c = jnp.dot(tri, oh[i * ck:(i + 1) * ck],
                    preferred_element_type=jnp.float32) + carry
        chunks.append(c)
        carry = c[ck - 1:ck, :]
    cum = jnp.concatenate(chunks, axis=0)
    counts = carry
    rank = jnp.sum((cum - oh) * oh, axis=1, keepdims=True)
    units_per = jnp.floor((counts + (_TM - 1)) * (1.0 / _TM))
    triu = (lax.broadcasted_iota(jnp.int32, (e, e), 0) <=
            lax.broadcasted_iota(jnp.int32, (e, e), 1)).astype(jnp.float32)
    cum_units = jnp.dot(units_per, triu, preferred_element_type=jnp.float32)
    unit_start = cum_units - units_per
    pos = (jnp.sum(oh * unit_start, axis=1, keepdims=True) * _TM + rank)
    pos_ref[...] = pos.astype(jnp.int32)
    del nu
    unit_start_ref[...] = unit_start.astype(jnp.int32)
    units_per_ref[...] = units_per.astype(jnp.int32)


def _moe_body(start_ref, nunit_ref, xs_ref, w1_ref, b1_ref, w2_ref, b2_ref,
              sw1_ref, sb1_ref, sw2_ref, sb2_ref, ys_ref):
    e = pl.program_id(0)
    s0 = start_ref[e]
    bm = 2 * _TM

    def unit(j, carry):
        blk = xs_ref[pl.ds(s0 * _TM + j * bm, bm), :]
        h = jnp.maximum(
            jnp.dot(blk, w1_ref[0], preferred_element_type=jnp.float32)
            + b1_ref[0], 0.0)
        y = jnp.dot(h, w2_ref[0], preferred_element_type=jnp.float32) + b2_ref[0]
        for s in range(sb1_ref.shape[0]):
            hs = jnp.maximum(
                jnp.dot(blk, sw1_ref[s], preferred_element_type=jnp.float32)
                + sb1_ref[s], 0.0)
            y = y + jnp.dot(hs, sw2_ref[s],
                            preferred_element_type=jnp.float32) + sb2_ref[s]
        ys_ref[pl.ds(s0 * _TM + j * bm, bm), :] = y
        return carry

    lax.fori_loop(0, (nunit_ref[e] + 1) // 2, unit, 0)


def _sc_scatter_rows(rows, idx2d, out_rows):
    t, d = rows.shape
    nw, b_per_w = idx2d.shape
    mesh = plsc.VectorSubcoreMesh(core_axis_name="c", subcore_axis_name="s")

    @functools.partial(
        pl.kernel, mesh=mesh,
        out_type=jax.ShapeDtypeStruct((out_rows, d), rows.dtype),
        scratch_types=[pltpu.VMEM((1, b_per_w), jnp.int32),
                       pltpu.VMEM((b_per_w, d), rows.dtype),
                       pltpu.SemaphoreType.DMA])
    def k(rows_hbm, idx_hbm, out_hbm, idx_v, rows_v, sem):
        wid = lax.axis_index("s") * 2 + lax.axis_index("c")
        pltpu.sync_copy(idx_hbm.at[pl.ds(wid, 1)], idx_v)
        pltpu.sync_copy(rows_hbm.at[pl.ds(wid * b_per_w, b_per_w)], rows_v)
        pltpu.async_copy(rows_v, out_hbm.at[idx_v.at[0]], sem).wait()

    return k(rows, idx2d)


def _sc_gather(table, idx):
    n_rows = idx.shape[0]
    d = table.shape[1]
    nw = 32
    b_per_w = n_rows // nw
    mesh = plsc.VectorSubcoreMesh(core_axis_name="c", subcore_axis_name="s")

    @functools.partial(
        pl.kernel, mesh=mesh,
        out_type=jax.ShapeDtypeStruct((n_rows, d), table.dtype),
        scratch_types=[pltpu.VMEM((b_per_w,), jnp.int32),
                       pltpu.VMEM((b_per_w, d), table.dtype),
                       pltpu.SemaphoreType.DMA])
    def k(table_hbm, idx_hbm, out_hbm, idx_v, rows_v, sem):
        wid = lax.axis_index("s") * 2 + lax.axis_index("c")
        base = wid * b_per_w
        pltpu.sync_copy(idx_hbm.at[pl.ds(base, b_per_w)], idx_v)
        pltpu.async_copy(table_hbm.at[idx_v], rows_v, sem).wait()
        pltpu.sync_copy(rows_v, out_hbm.at[pl.ds(base, b_per_w)])

    return k(table, idx)


def kernel(x, router_w, router_b, w1, b1, w2, b2, sw1, sb1, sw2, sb2):
    bsz, seq, hdim = x.shape
    flat = x.reshape(-1, hdim)
    t = flat.shape[0]
    n_exp = w1.shape[0]
    n_units = (t // _TM) + n_exp
    p = n_units * _TM

    logits, pos2, unit_start, units_per = pl.pallas_call(
        _router_body,
        grid=(1,),
        in_specs=[pl.BlockSpec((t, hdim), lambda i: (0, 0)),
                  pl.BlockSpec((hdim, n_exp), lambda i: (0, 0)),
                  pl.BlockSpec((1, n_exp), lambda i: (0, 0))],
        out_specs=[pl.BlockSpec((t, n_exp), lambda i: (0, 0)),
                   pl.BlockSpec((t, 1), lambda i: (0, 0)),
                   pl.BlockSpec((1, n_exp), lambda i: (0, 0)),
                   pl.BlockSpec((1, n_exp), lambda i: (0, 0))],
        out_shape=[jax.ShapeDtypeStruct((t, n_exp), jnp.float32),
                   jax.ShapeDtypeStruct((t, 1), jnp.int32),
                   jax.ShapeDtypeStruct((1, n_exp), jnp.int32),
                   jax.ShapeDtypeStruct((1, n_exp), jnp.int32)],
    )(flat, router_w, router_b.reshape(1, n_exp))
    pos = pos2.reshape(t)

    xs = _sc_scatter_rows(flat, pos.reshape(32, t // 32), p)

    grid_spec = pltpu.PrefetchScalarGridSpec(
        num_scalar_prefetch=2,
        grid=(n_exp,),
        in_specs=[pl.BlockSpec((p, hdim), lambda e, s, n: (0, 0)),
                  pl.BlockSpec((1, hdim, hdim), lambda e, s, n: (e, 0, 0)),
                  pl.BlockSpec((1, 1, hdim), lambda e, s, n: (e, 0, 0)),
                  pl.BlockSpec((1, hdim, hdim), lambda e, s, n: (e, 0, 0)),
                  pl.BlockSpec((1, 1, hdim), lambda e, s, n: (e, 0, 0)),
                  pl.BlockSpec(sw1.shape, lambda e, s, n: (0, 0, 0)),
                  pl.BlockSpec((sb1.shape[0], 1, hdim), lambda e, s, n: (0, 0, 0)),
                  pl.BlockSpec(sw2.shape, lambda e, s, n: (0, 0, 0)),
                  pl.BlockSpec((sb2.shape[0], 1, hdim), lambda e, s, n: (0, 0, 0))],
        out_specs=pl.BlockSpec((p, hdim), lambda e, s, n: (0, 0)),
    )
    nsh = sw1.shape[0]
    ys = pl.pallas_call(
        _moe_body, grid_spec=grid_spec,
        out_shape=jax.ShapeDtypeStruct((p, hdim), jnp.float32),
    )(unit_start.reshape(n_exp), units_per.reshape(n_exp),
      xs, w1, b1.reshape(n_exp, 1, hdim), w2, b2.reshape(n_exp, 1, hdim),
      sw1, sb1.reshape(nsh, 1, hdim), sw2, sb2.reshape(nsh, 1, hdim))

    out = _sc_gather(ys, pos).reshape(bsz, seq, hdim)
    return (out, logits)
